# Initial kernel scaffold; baseline (speedup 1.0000x reference)
import jax
import jax.numpy as jnp
from jax.experimental import pallas as pl


def kernel(idxs, vals, emb_tables, first_w):
    raise NotImplementedError("write your pallas kernel here")


# speedup vs baseline: 6.7527x; 6.7527x over previous
import jax
import jax.numpy as jnp
from jax import lax
from jax.experimental import pallas as pl
from jax.experimental.pallas import tpu as pltpu, tpu_sc as plsc

V = 100000
F = 26
D = 16
B = 4096
G = 25
GRID = G * G
GRID_PAD = 640
N_DMA = GRID_PAD // 128
FO_PAD = 32

NC, NS = 2, 16
NW = NC * NS
B_PER_W = B // NW
CB = 8
N_SUB = B_PER_W // CB


def _ffm_body(idx_hbm, foidx_hbm, vb_hbm, emb_hbm, fw_hbm, out_hbm,
              idx_v, foidx_v, vb_v, rows_v, fo_rows_v, out_v, sem):
    wid = lax.axis_index("s") * NC + lax.axis_index("c")
    base = wid * B_PER_W
    lane = lax.broadcasted_iota(jnp.int32, (D,), 0)

    def sub_chunk(c, _):
        b0 = base + c * CB
        pltpu.sync_copy(idx_hbm.at[pl.ds(b0, CB)], idx_v)
        pltpu.sync_copy(foidx_hbm.at[pl.ds(b0, CB)], foidx_v)
        pltpu.sync_copy(vb_hbm.at[pl.ds(b0, CB)], vb_v)

        def per_b(bb, res):
            copies = [
                pltpu.async_copy(
                    emb_hbm.at[idx_v.at[bb, g]],
                    rows_v.at[bb, pl.ds(g * 128, 128)],
                    sem,
                )
                for g in range(N_DMA)
            ]
            copies.append(
                pltpu.async_copy(fw_hbm.at[foidx_v.at[bb]], fo_rows_v.at[bb], sem)
            )
            for cp in copies:
                cp.wait()

            def fo_step(j, acc):
                return acc + fo_rows_v[bb, j, :] * vb_v[bb, j, :]

            facc = lax.fori_loop(0, F, fo_step, jnp.zeros((D,), jnp.float32))

            def outer(i, acc):
                pvi = vb_v[bb, i, :]

                def inner(j, acc):
                    a = rows_v[bb, i * G + j, :]
                    b = rows_v[bb, j * G + i, :]
                    return acc + a * b * pvi * vb_v[bb, j, :]

                return lax.fori_loop(i + 1, G, inner, acc)

            acc = lax.fori_loop(0, G, outer, facc)
            for sh in (8, 4, 2, 1):
                acc = acc + acc.at[lane ^ sh].get(mode="promise_in_bounds")
            return jnp.where(lane == bb, acc, res)

        res = lax.fori_loop(0, CB, per_b, jnp.zeros((D,), jnp.float32))
        out_v[pl.ds(c * CB, D)] = res
        return 0

    lax.fori_loop(0, N_SUB, sub_chunk, 0)

    def sig(k, _):
        x = out_v[pl.ds(k * D, D)]
        out_v[pl.ds(k * D, D)] = 1.0 / (1.0 + jnp.exp(-x))
        return 0

    lax.fori_loop(0, B_PER_W // D, sig, 0)
    pltpu.sync_copy(out_v.at[pl.ds(0, B_PER_W)], out_hbm.at[pl.ds(base, B_PER_W)])


@jax.jit
def _ffm_call(idx_grid, fo_idx, vals_b, emb_flat, fw_pad):
    mesh = plsc.VectorSubcoreMesh(core_axis_name="c", subcore_axis_name="s")
    return pl.kernel(
        _ffm_body,
        out_type=jax.ShapeDtypeStruct((B,), jnp.float32),
        mesh=mesh,
        compiler_params=pltpu.CompilerParams(use_tc_tiling_on_sc=False),
        scratch_types=[
            pltpu.VMEM((CB, N_DMA, 128), jnp.int32),
            pltpu.VMEM((CB, FO_PAD), jnp.int32),
            pltpu.VMEM((CB, F, D), jnp.float32),
            pltpu.VMEM((CB, GRID_PAD, D), jnp.float32),
            pltpu.VMEM((CB, FO_PAD, D), jnp.float32),
            pltpu.VMEM((B_PER_W + D,), jnp.float32),
            pltpu.SemaphoreType.DMA,
        ],
    )(idx_grid, fo_idx, vals_b, emb_flat, fw_pad)


def kernel(idxs, vals, emb_tables, first_w):
    emb_flat = emb_tables.reshape(F * V, D)
    fw_pad = jnp.pad(first_w, ((0, 0), (0, D - 1)))
    ii = (jnp.arange(G, dtype=jnp.int32) * V)[None, :, None]
    grid = (ii + idxs[:, None, :G]).reshape(B, GRID)
    idx_grid = jnp.pad(grid, ((0, 0), (0, GRID_PAD - GRID))).reshape(B, N_DMA, 128)
    fo_idx = jnp.pad(idxs, ((0, 0), (0, FO_PAD - F)))
    vals_b = jnp.broadcast_to(vals[:, :, None], (B, F, D))
    return _ffm_call(idx_grid, fo_idx, vals_b, emb_flat, fw_pad)
